# baseline (speedup 1.0000x reference)
import functools

import jax
import jax.numpy as jnp
import numpy as np
from jax import lax
from jax.experimental import pallas as pl
from jax.experimental.pallas import tpu as pltpu
from jax.experimental.pallas import tpu_sc as plsc

N = 10000
E = 320000
NG = 100

NW = 32
CH = 1024
EW = 10240
EP = NW * EW
NCH = EW // CH
NP = 10240
NSUB = NP // 16

_SC_MESH = dict(core_axis_name="c", subcore_axis_name="s",
                num_cores=2, num_subcores=16)


CHG = 512
NCHG = EW // CHG


def _gather_body(table, srcv, out, idx_v, rows_v, gsem):
    wid = lax.axis_index("s") * 2 + lax.axis_index("c")

    @pl.loop(0, NCH)
    def _chunk(ci):
        irow = wid * (EW // 128) + ci * 8
        pltpu.sync_copy(srcv.at[pl.ds(irow, 8)], idx_v)
        for half in range(2):
            descs = []
            for j in range(4):
                descs.append(
                    pltpu.async_copy(table.at[idx_v.at[half * 4 + j]],
                                     rows_v.at[pl.ds(j * 128, 128)], gsem))
            for d in descs:
                d.wait()
            base = wid * EW + ci * CH + half * CHG
            pltpu.sync_copy(rows_v, out.at[pl.ds(base, CHG)])


def _sc_gather(table, srcv):
    return pl.kernel(
        _gather_body,
        out_type=jax.ShapeDtypeStruct((EP, 128), jnp.float32),
        mesh=plsc.VectorSubcoreMesh(**_SC_MESH),
        scratch_types=[
            pltpu.VMEM((8, 128), jnp.int32),
            pltpu.VMEM((CHG, 128), jnp.float32),
            pltpu.SemaphoreType.DMA,
        ],
    )(table, srcv)


CHS = 256


def _scatter_body(msgv, dstv, zer, out, idx_v, rows_v, acc):
    cid = lax.axis_index("c")
    sid = lax.axis_index("s")
    wid = sid * 2 + cid

    pltpu.sync_copy(zer, acc.at[pl.ds(sid * NSUB, NSUB)])
    plsc.subcore_barrier()

    @pl.loop(0, NCH)
    def _chunk(ci):
        irow = wid * (EW // 128) + ci * 8
        pltpu.sync_copy(dstv.at[pl.ds(irow, 8)], idx_v)
        for q in range(4):
            base = wid * EW + ci * CH + q * CHS
            pltpu.sync_copy(msgv.at[pl.ds(base, CHS)], rows_v)
            for j in range(2):
                pltpu.sync_copy(rows_v.at[pl.ds(j * 128, 128)],
                                acc.at[idx_v.at[q * 2 + j]], add=True)

    plsc.subcore_barrier()
    pltpu.sync_copy(acc.at[pl.ds(sid * NSUB, NSUB)],
                    out.at[cid].at[pl.ds(sid * NSUB, NSUB)])


def _sc_scatter(msgv, dstv, zer, F):
    return pl.kernel(
        _scatter_body,
        out_type=jax.ShapeDtypeStruct((2, NP, F), jnp.float32),
        mesh=plsc.VectorSubcoreMesh(**_SC_MESH),
        scratch_types=[
            pltpu.VMEM((8, 128), jnp.int32),
            pltpu.VMEM((CHS, F), jnp.float32),
            pltpu.VMEM_SHARED((NP, F), jnp.float32),
        ],
        compiler_params=pltpu.CompilerParams(use_tc_tiling_on_sc=False),
    )(msgv, dstv, zer)


_BE = 2048


def _dense_body(out_D, F_in, xs_ref, th_ref, pc_ref, wk_ref, u0_ref, u1_ref,
                v0_ref, v1_ref, msg_ref):
    B = xs_ref.shape[0]
    xs = xs_ref[:, :F_in]
    th = th_ref[...]
    c1 = jnp.cos(th)
    s1 = jnp.sin(th)
    c2 = 2.0 * c1 * c1 - 1.0
    s2 = 2.0 * s1 * c1
    dn = (((1,), (0,)), ((), ()))
    Y = lax.dot_general(xs, wk_ref[...], dn, preferred_element_type=jnp.float32)
    T = (Y[:, 0:32] + c1 * Y[:, 32:64] + s1 * Y[:, 64:96]
         + c2 * Y[:, 96:128] + s2 * Y[:, 128:160])
    P = pc_ref[...]
    if out_D == 5:
        msg = (lax.dot_general(T, u0_ref[...], dn, preferred_element_type=jnp.float32)
               * lax.dot_general(P, v0_ref[...], dn, preferred_element_type=jnp.float32)
               + lax.dot_general(T, u1_ref[...], dn, preferred_element_type=jnp.float32)
               * lax.dot_general(P, v1_ref[...], dn, preferred_element_type=jnp.float32))
    else:
        msg = T[:, :16] * P[:, 0:1] + T[:, 16:32] * P[:, 5:6]
    eid = pl.program_id(0) * B + lax.broadcasted_iota(jnp.int32, (B, 1), 0)
    msg_ref[...] = jnp.where(eid < E, msg, 0.0)


def _dense_body_d1(xs_ref, pc_ref, wf_ref, msg_ref):
    B = xs_ref.shape[0]
    dn = (((1,), (0,)), ((), ()))
    T = lax.dot_general(xs_ref[:, :16], wf_ref[...], dn,
                        preferred_element_type=jnp.float32)
    P = pc_ref[...]
    msg = T[:, :16] * P[:, 0:1] + T[:, 16:32] * P[:, 5:6]
    eid = pl.program_id(0) * B + lax.broadcasted_iota(jnp.int32, (B, 1), 0)
    msg_ref[...] = jnp.where(eid < E, msg, 0.0)


def _uv_consts():
    u0 = np.zeros((32, 80), np.float32)
    u1 = np.zeros((32, 80), np.float32)
    v0 = np.zeros((10, 80), np.float32)
    v1 = np.zeros((10, 80), np.float32)
    for o in range(16):
        for p in range(5):
            u0[o, o * 5 + p] = 1.0
            u1[16 + o, o * 5 + p] = 1.0
            v0[p, o * 5 + p] = 1.0
            v1[5 + p, o * 5 + p] = 1.0
    return tuple(jnp.asarray(a) for a in (u0, u1, v0, v1))


def _tc_dense(xs, th, pc, wk, F_in, out_D):
    F_out = 16 * out_D
    grid = EP // _BE
    eb = lambda c: pl.BlockSpec((_BE, c), lambda i: (i, 0))
    wb = lambda a: pl.BlockSpec(a.shape, lambda i: (0, 0))
    uv = _uv_consts()
    return pl.pallas_call(
        functools.partial(_dense_body, out_D, F_in),
        grid=(grid,),
        in_specs=[eb(128), eb(1), eb(10), wb(wk)] + [wb(a) for a in uv],
        out_specs=eb(F_out),
        out_shape=jax.ShapeDtypeStruct((EP, F_out), jnp.float32),
    )(xs, th, pc, wk, *uv)


def _tc_dense_d1(xs, pc, wf):
    grid = EP // _BE
    eb = lambda c: pl.BlockSpec((_BE, c), lambda i: (i, 0))
    return pl.pallas_call(
        _dense_body_d1,
        grid=(grid,),
        in_specs=[eb(128), eb(10), pl.BlockSpec(wf.shape, lambda i: (0, 0))],
        out_specs=eb(16),
        out_shape=jax.ShapeDtypeStruct((EP, 16), jnp.float32),
    )(xs, pc, wf)


def _epi_body(F, acc_ref, b_ref, sc_ref, m_ref, h_ref):
    h = acc_ref[0, :N] + acc_ref[1, :N] + b_ref[...]
    if m_ref is not None:
        dn = (((1,), (0,)), ((), ()))
        h = h + lax.dot_general(sc_ref[...], m_ref[...], dn,
                                preferred_element_type=jnp.float32)
    elif sc_ref is not None:
        h = h + sc_ref[:, :F]
    h = jnp.maximum(h, 0.0)
    h_ref[:, :F] = h
    h_ref[:, F:] = jnp.zeros((N, 128 - F), jnp.float32)


def _tc_epilogue(acc, bias, F, sc=None, m=None):
    args = [acc, bias.reshape(1, F)]
    if sc is not None:
        args.append(sc)
    if m is not None:
        args.append(m)

    def body(*refs):
        acc_ref, b_ref = refs[0], refs[1]
        sc_ref = refs[2] if sc is not None else None
        m_ref = refs[3] if m is not None else None
        _epi_body(F, acc_ref, b_ref, sc_ref, m_ref, refs[-1])

    return pl.pallas_call(
        body,
        out_shape=jax.ShapeDtypeStruct((N, 128), jnp.float32),
    )(*args)


def _final_body(acc_ref, b_ref, xb_ref, m_ref, w1_ref, b1_ref, w2_ref,
                b2_ref, batch_ref, out_ref):
    dn = (((1,), (0,)), ((), ()))
    dnt = (((1,), (1,)), ((), ()))
    h = acc_ref[0, :N] + acc_ref[1, :N] + b_ref[...]
    h = h + lax.dot_general(xb_ref[...], m_ref[...], dn,
                            preferred_element_type=jnp.float32)
    h = jnp.maximum(h, 0.0)
    h1 = jnp.maximum(
        lax.dot_general(h, w1_ref[...], dnt,
                        preferred_element_type=jnp.float32) + b1_ref[...], 0.0)
    h2 = lax.dot_general(h1, w2_ref[...], dnt,
                         preferred_element_type=jnp.float32) + b2_ref[...]
    seg = lax.broadcasted_iota(jnp.int32, (NG, N), 0)
    onehot = jnp.where(batch_ref[...] == seg, 1.0, 0.0)
    sums = lax.dot_general(onehot, h2, dn, preferred_element_type=jnp.float32)
    cnt = jnp.sum(onehot, axis=1, keepdims=True)
    pooled = sums / jnp.maximum(cnt, 1.0)
    mx = jnp.max(pooled, axis=1, keepdims=True)
    lse = mx + jnp.log(jnp.sum(jnp.exp(pooled - mx), axis=1, keepdims=True))
    out_ref[...] = pooled - lse


def _tc_final(acc, b16, xb3, m3, Wl1, bl1, Wl2, bl2, batch):
    return pl.pallas_call(
        _final_body,
        out_shape=jax.ShapeDtypeStruct((NG, 40), jnp.float32),
    )(acc, b16.reshape(1, 16), xb3, m3, Wl1, bl1.reshape(1, 256), Wl2,
      bl2.reshape(1, 40), batch.reshape(1, N).astype(jnp.int32))


def _rot_basis(C_in, F_in):
    M = [np.zeros((F_in, F_in), np.float32) for _ in range(5)]
    for i in range(C_in):
        o = i * 5
        M[0][o, o] = 1.0
        M[1][o + 1, o + 1] = M[1][o + 2, o + 2] = 1.0
        M[2][o + 2, o + 1] = -1.0
        M[2][o + 1, o + 2] = 1.0
        M[3][o + 3, o + 3] = M[3][o + 4, o + 4] = 1.0
        M[4][o + 4, o + 3] = -1.0
        M[4][o + 3, o + 4] = 1.0
    return M


def _wk(W, C_in, F_in):
    wf = _wflat(W, F_in)
    M = _rot_basis(C_in, F_in)
    return jnp.concatenate([jnp.asarray(m) @ wf for m in M], axis=1)


def _wflat(W, F_in):
    NR, CO, CI, D = W.shape
    wf = jnp.transpose(W, (2, 3, 0, 1)).reshape(CI * D, NR * CO)
    if CI * D < F_in:
        wf = jnp.pad(wf, ((0, F_in - CI * D), (0, 0)))
    return wf.astype(jnp.float32)


def _shortcut_mat(Ws, D):
    CO, CI = Ws.shape
    m = jnp.zeros((CI * 5, CO * D), jnp.float32)
    for p in range(D):
        m = m.at[p::5, p::D].set(jnp.transpose(Ws))
    return m


def kernel(x, edge_index, precomp, connection, batch, W1a, b1a, W1b, b1b, Ws1,
           W2a, b2a, W2b, b2b, W3a, b3a, W3b, b3b, Ws3, Wl1, bl1, Wl2, bl2):
    f32 = jnp.float32
    pad = EP - E
    fill = (jnp.arange(pad, dtype=jnp.int32) % N)
    srcv = jnp.concatenate([edge_index[0].astype(jnp.int32), fill]).reshape(EP // 128, 128)
    dstv = jnp.concatenate([edge_index[1].astype(jnp.int32), fill]).reshape(EP // 128, 128)
    th = jnp.pad(connection.astype(f32), (0, pad)).reshape(EP, 1)
    pc = jnp.pad(precomp.astype(f32).reshape(E, 10), ((0, pad), (0, 0)))

    z80 = jnp.zeros((NSUB, 80), f32)
    z16 = jnp.zeros((NSUB, 16), f32)

    def conv(table, F_in, wk, out_D):
        xs = _sc_gather(table, srcv)
        if wk.shape[1] == 32:
            msg = _tc_dense_d1(xs, pc, wk)
        else:
            msg = _tc_dense(xs, th, pc, wk, F_in, out_D)
        F_out = 16 * out_D
        return _sc_scatter(msg, dstv, z80 if F_out == 80 else z16, F_out)

    def bias80(b):
        return jnp.repeat(b.astype(f32), 5) * jnp.tile(jnp.asarray([1., 0, 0, 0, 0]), 16)

    t0 = jnp.pad(x.astype(f32).reshape(N, 35), ((0, 0), (0, 93)))

    a = conv(t0, 40, _wk(W1a, 7, 40), 5)
    h1 = _tc_epilogue(a, bias80(b1a), 80)
    a = conv(h1, 80, _wk(W1b, 16, 80), 5)
    hb1 = _tc_epilogue(a, bias80(b1b), 80, sc=t0,
                       m=jnp.pad(_shortcut_mat(Ws1, 5), ((0, 93), (0, 0))))

    a = conv(hb1, 80, _wk(W2a, 16, 80), 5)
    h2 = _tc_epilogue(a, bias80(b2a), 80)
    a = conv(h2, 80, _wk(W2b, 16, 80), 5)
    hb2 = _tc_epilogue(a, bias80(b2b), 80, sc=hb1)

    a = conv(hb2, 80, _wk(W3a, 16, 80), 1)
    h3 = _tc_epilogue(a, b3a.astype(f32), 16)
    a = conv(h3, 16, _wflat(W3b, 16), 1)

    m3 = jnp.pad(_shortcut_mat(Ws3, 1), ((0, 48), (0, 0)))
    return _tc_final(a, b3b.astype(f32), hb2, m3, Wl1.astype(f32), bl1,
                     Wl2.astype(f32), bl2, batch)


# speedup vs baseline: 1.0877x; 1.0877x over previous
import functools

import jax
import jax.numpy as jnp
import numpy as np
from jax import lax
from jax.experimental import pallas as pl
from jax.experimental.pallas import tpu as pltpu
from jax.experimental.pallas import tpu_sc as plsc

N = 10000
E = 320000
NG = 100

NW = 32
CH = 1024
EW = 10240
EP = NW * EW
NCH = EW // CH
NP = 10240
NSUB = NP // 16

_SC_MESH = dict(core_axis_name="c", subcore_axis_name="s",
                num_cores=2, num_subcores=16)


CHG = 512
NCHG = EW // CHG


def _gather_body(table, srcv, out, idx_v, rows_v, gsem):
    wid = lax.axis_index("s") * 2 + lax.axis_index("c")

    @pl.loop(0, NCH)
    def _chunk(ci):
        irow = wid * (EW // 128) + ci * 8
        pltpu.sync_copy(srcv.at[pl.ds(irow, 8)], idx_v)
        for half in range(2):
            descs = []
            for j in range(4):
                descs.append(
                    pltpu.async_copy(table.at[idx_v.at[half * 4 + j]],
                                     rows_v.at[pl.ds(j * 128, 128)], gsem))
            for d in descs:
                d.wait()
            base = wid * EW + ci * CH + half * CHG
            pltpu.sync_copy(rows_v, out.at[pl.ds(base, CHG)])


def _sc_gather(table, srcv):
    return pl.kernel(
        _gather_body,
        out_type=jax.ShapeDtypeStruct((EP, 128), jnp.float32),
        mesh=plsc.VectorSubcoreMesh(**_SC_MESH),
        scratch_types=[
            pltpu.VMEM((8, 128), jnp.int32),
            pltpu.VMEM((CHG, 128), jnp.float32),
            pltpu.SemaphoreType.DMA,
        ],
    )(table, srcv)


CHS = 256


def _scatter_body(msgv, dstv, zer, out, idx_v, rows_v, acc):
    cid = lax.axis_index("c")
    sid = lax.axis_index("s")
    wid = sid * 2 + cid

    pltpu.sync_copy(zer, acc.at[pl.ds(sid * NSUB, NSUB)])
    plsc.subcore_barrier()

    @pl.loop(0, NCH)
    def _chunk(ci):
        irow = wid * (EW // 128) + ci * 8
        pltpu.sync_copy(dstv.at[pl.ds(irow, 8)], idx_v)
        for q in range(4):
            base = wid * EW + ci * CH + q * CHS
            pltpu.sync_copy(msgv.at[pl.ds(base, CHS)], rows_v)
            for j in range(2):
                pltpu.sync_copy(rows_v.at[pl.ds(j * 128, 128)],
                                acc.at[idx_v.at[q * 2 + j]], add=True)

    plsc.subcore_barrier()
    pltpu.sync_copy(acc.at[pl.ds(sid * NSUB, NSUB)],
                    out.at[cid].at[pl.ds(sid * NSUB, NSUB)])


def _sc_scatter(msgv, dstv, zer):
    return pl.kernel(
        _scatter_body,
        out_type=jax.ShapeDtypeStruct((2, NP, 128), jnp.float32),
        mesh=plsc.VectorSubcoreMesh(**_SC_MESH),
        scratch_types=[
            pltpu.VMEM((8, 128), jnp.int32),
            pltpu.VMEM((CHS, 128), jnp.float32),
            pltpu.VMEM_SHARED((NP, 128), jnp.float32),
        ],
    )(msgv, dstv, zer)


_BE = 2048


def _dense_body(out_D, F_in, xs_ref, th_ref, pc_ref, wk_ref, u0_ref, u1_ref,
                v0_ref, v1_ref, msg_ref):
    B = xs_ref.shape[0]
    xs = xs_ref[:, :F_in]
    th = th_ref[...]
    c1 = jnp.cos(th)
    s1 = jnp.sin(th)
    c2 = 2.0 * c1 * c1 - 1.0
    s2 = 2.0 * s1 * c1
    dn = (((1,), (0,)), ((), ()))
    Y = lax.dot_general(xs, wk_ref[...], dn, preferred_element_type=jnp.float32)
    T = (Y[:, 0:32] + c1 * Y[:, 32:64] + s1 * Y[:, 64:96]
         + c2 * Y[:, 96:128] + s2 * Y[:, 128:160])
    P = pc_ref[...]
    if out_D == 5:
        msg = (lax.dot_general(T, u0_ref[...], dn, preferred_element_type=jnp.float32)
               * lax.dot_general(P, v0_ref[...], dn, preferred_element_type=jnp.float32)
               + lax.dot_general(T, u1_ref[...], dn, preferred_element_type=jnp.float32)
               * lax.dot_general(P, v1_ref[...], dn, preferred_element_type=jnp.float32))
    else:
        msg = T[:, :16] * P[:, 0:1] + T[:, 16:32] * P[:, 5:6]
    eid = pl.program_id(0) * B + lax.broadcasted_iota(jnp.int32, (B, 1), 0)
    F_out = msg.shape[1]
    msg_ref[:, :F_out] = jnp.where(eid < E, msg, 0.0)
    msg_ref[:, F_out:] = jnp.zeros((B, 128 - F_out), jnp.float32)


def _dense_body_d1(xs_ref, pc_ref, wf_ref, msg_ref):
    B = xs_ref.shape[0]
    dn = (((1,), (0,)), ((), ()))
    T = lax.dot_general(xs_ref[:, :16], wf_ref[...], dn,
                        preferred_element_type=jnp.float32)
    P = pc_ref[...]
    msg = T[:, :16] * P[:, 0:1] + T[:, 16:32] * P[:, 5:6]
    eid = pl.program_id(0) * B + lax.broadcasted_iota(jnp.int32, (B, 1), 0)
    msg_ref[:, :16] = jnp.where(eid < E, msg, 0.0)
    msg_ref[:, 16:] = jnp.zeros((B, 112), jnp.float32)


def _uv_consts():
    u0 = np.zeros((32, 80), np.float32)
    u1 = np.zeros((32, 80), np.float32)
    v0 = np.zeros((10, 80), np.float32)
    v1 = np.zeros((10, 80), np.float32)
    for o in range(16):
        for p in range(5):
            u0[o, o * 5 + p] = 1.0
            u1[16 + o, o * 5 + p] = 1.0
            v0[p, o * 5 + p] = 1.0
            v1[5 + p, o * 5 + p] = 1.0
    return tuple(jnp.asarray(a) for a in (u0, u1, v0, v1))


def _tc_dense(xs, th, pc, wk, F_in, out_D):
    F_out = 16 * out_D
    grid = EP // _BE
    eb = lambda c: pl.BlockSpec((_BE, c), lambda i: (i, 0))
    wb = lambda a: pl.BlockSpec(a.shape, lambda i: (0, 0))
    uv = _uv_consts()
    return pl.pallas_call(
        functools.partial(_dense_body, out_D, F_in),
        grid=(grid,),
        in_specs=[eb(128), eb(1), eb(10), wb(wk)] + [wb(a) for a in uv],
        out_specs=eb(128),
        out_shape=jax.ShapeDtypeStruct((EP, 128), jnp.float32),
    )(xs, th, pc, wk, *uv)


def _tc_dense_d1(xs, pc, wf):
    grid = EP // _BE
    eb = lambda c: pl.BlockSpec((_BE, c), lambda i: (i, 0))
    return pl.pallas_call(
        _dense_body_d1,
        grid=(grid,),
        in_specs=[eb(128), eb(10), pl.BlockSpec(wf.shape, lambda i: (0, 0))],
        out_specs=eb(128),
        out_shape=jax.ShapeDtypeStruct((EP, 128), jnp.float32),
    )(xs, pc, wf)


def _epi_body(F, acc_ref, b_ref, sc_ref, m_ref, h_ref):
    h = acc_ref[0, :N, :F] + acc_ref[1, :N, :F] + b_ref[...]
    if m_ref is not None:
        dn = (((1,), (0,)), ((), ()))
        h = h + lax.dot_general(sc_ref[...], m_ref[...], dn,
                                preferred_element_type=jnp.float32)
    elif sc_ref is not None:
        h = h + sc_ref[:, :F]
    h = jnp.maximum(h, 0.0)
    h_ref[:, :F] = h
    h_ref[:, F:] = jnp.zeros((N, 128 - F), jnp.float32)


def _tc_epilogue(acc, bias, F, sc=None, m=None):
    args = [acc, bias.reshape(1, F)]
    if sc is not None:
        args.append(sc)
    if m is not None:
        args.append(m)

    def body(*refs):
        acc_ref, b_ref = refs[0], refs[1]
        sc_ref = refs[2] if sc is not None else None
        m_ref = refs[3] if m is not None else None
        _epi_body(F, acc_ref, b_ref, sc_ref, m_ref, refs[-1])

    return pl.pallas_call(
        body,
        out_shape=jax.ShapeDtypeStruct((N, 128), jnp.float32),
    )(*args)


def _final_body(acc_ref, b_ref, xb_ref, m_ref, w1_ref, b1_ref, w2_ref,
                b2_ref, batch_ref, out_ref):
    dn = (((1,), (0,)), ((), ()))
    dnt = (((1,), (1,)), ((), ()))
    h = acc_ref[0, :N, :16] + acc_ref[1, :N, :16] + b_ref[...]
    h = h + lax.dot_general(xb_ref[...], m_ref[...], dn,
                            preferred_element_type=jnp.float32)
    h = jnp.maximum(h, 0.0)
    h1 = jnp.maximum(
        lax.dot_general(h, w1_ref[...], dnt,
                        preferred_element_type=jnp.float32) + b1_ref[...], 0.0)
    h2 = lax.dot_general(h1, w2_ref[...], dnt,
                         preferred_element_type=jnp.float32) + b2_ref[...]
    seg = lax.broadcasted_iota(jnp.int32, (NG, N), 0)
    onehot = jnp.where(batch_ref[...] == seg, 1.0, 0.0)
    sums = lax.dot_general(onehot, h2, dn, preferred_element_type=jnp.float32)
    cnt = jnp.sum(onehot, axis=1, keepdims=True)
    pooled = sums / jnp.maximum(cnt, 1.0)
    mx = jnp.max(pooled, axis=1, keepdims=True)
    lse = mx + jnp.log(jnp.sum(jnp.exp(pooled - mx), axis=1, keepdims=True))
    out_ref[...] = pooled - lse


def _tc_final(acc, b16, xb3, m3, Wl1, bl1, Wl2, bl2, batch):
    return pl.pallas_call(
        _final_body,
        out_shape=jax.ShapeDtypeStruct((NG, 40), jnp.float32),
    )(acc, b16.reshape(1, 16), xb3, m3, Wl1, bl1.reshape(1, 256), Wl2,
      bl2.reshape(1, 40), batch.reshape(1, N).astype(jnp.int32))


def _rot_basis(C_in, F_in):
    M = [np.zeros((F_in, F_in), np.float32) for _ in range(5)]
    for i in range(C_in):
        o = i * 5
        M[0][o, o] = 1.0
        M[1][o + 1, o + 1] = M[1][o + 2, o + 2] = 1.0
        M[2][o + 2, o + 1] = -1.0
        M[2][o + 1, o + 2] = 1.0
        M[3][o + 3, o + 3] = M[3][o + 4, o + 4] = 1.0
        M[4][o + 4, o + 3] = -1.0
        M[4][o + 3, o + 4] = 1.0
    return M


def _wk(W, C_in, F_in):
    wf = _wflat(W, F_in)
    M = _rot_basis(C_in, F_in)
    return jnp.concatenate([jnp.asarray(m) @ wf for m in M], axis=1)


def _wflat(W, F_in):
    NR, CO, CI, D = W.shape
    wf = jnp.transpose(W, (2, 3, 0, 1)).reshape(CI * D, NR * CO)
    if CI * D < F_in:
        wf = jnp.pad(wf, ((0, F_in - CI * D), (0, 0)))
    return wf.astype(jnp.float32)


def _shortcut_mat(Ws, D):
    CO, CI = Ws.shape
    m = jnp.zeros((CI * 5, CO * D), jnp.float32)
    for p in range(D):
        m = m.at[p::5, p::D].set(jnp.transpose(Ws))
    return m


def kernel(x, edge_index, precomp, connection, batch, W1a, b1a, W1b, b1b, Ws1,
           W2a, b2a, W2b, b2b, W3a, b3a, W3b, b3b, Ws3, Wl1, bl1, Wl2, bl2):
    f32 = jnp.float32
    pad = EP - E
    fill = (jnp.arange(pad, dtype=jnp.int32) % N)
    srcv = jnp.concatenate([edge_index[0].astype(jnp.int32), fill]).reshape(EP // 128, 128)
    dstv = jnp.concatenate([edge_index[1].astype(jnp.int32), fill]).reshape(EP // 128, 128)
    th = jnp.pad(connection.astype(f32), (0, pad)).reshape(EP, 1)
    pc = jnp.pad(precomp.astype(f32).reshape(E, 10), ((0, pad), (0, 0)))

    z128 = jnp.zeros((NSUB, 128), f32)

    def conv(table, F_in, wk, out_D):
        xs = _sc_gather(table, srcv)
        if wk.shape[1] == 32:
            msg = _tc_dense_d1(xs, pc, wk)
        else:
            msg = _tc_dense(xs, th, pc, wk, F_in, out_D)
        return _sc_scatter(msg, dstv, z128)

    def bias80(b):
        return jnp.repeat(b.astype(f32), 5) * jnp.tile(jnp.asarray([1., 0, 0, 0, 0]), 16)

    t0 = jnp.pad(x.astype(f32).reshape(N, 35), ((0, 0), (0, 93)))

    a = conv(t0, 40, _wk(W1a, 7, 40), 5)
    h1 = _tc_epilogue(a, bias80(b1a), 80)
    a = conv(h1, 80, _wk(W1b, 16, 80), 5)
    hb1 = _tc_epilogue(a, bias80(b1b), 80, sc=t0,
                       m=jnp.pad(_shortcut_mat(Ws1, 5), ((0, 93), (0, 0))))

    a = conv(hb1, 80, _wk(W2a, 16, 80), 5)
    h2 = _tc_epilogue(a, bias80(b2a), 80)
    a = conv(h2, 80, _wk(W2b, 16, 80), 5)
    hb2 = _tc_epilogue(a, bias80(b2b), 80, sc=hb1)

    a = conv(hb2, 80, _wk(W3a, 16, 80), 1)
    h3 = _tc_epilogue(a, b3a.astype(f32), 16)
    a = conv(h3, 16, _wflat(W3b, 16), 1)

    m3 = jnp.pad(_shortcut_mat(Ws3, 1), ((0, 48), (0, 0)))
    return _tc_final(a, b3b.astype(f32), hb2, m3, Wl1.astype(f32), bl1,
                     Wl2.astype(f32), bl2, batch)


# speedup vs baseline: 1.1919x; 1.0958x over previous
import functools

import jax
import jax.numpy as jnp
import numpy as np
from jax import lax
from jax.experimental import pallas as pl
from jax.experimental.pallas import tpu as pltpu
from jax.experimental.pallas import tpu_sc as plsc

N = 10000
E = 320000
NG = 100

NW = 32
CH = 1024
EW = 10240
EP = NW * EW
NCH = EW // CH
NP = 10240
NSUB = NP // 16

_SC_MESH = dict(core_axis_name="c", subcore_axis_name="s",
                num_cores=2, num_subcores=16)


CHG = 512
NCHG = EW // CHG


def _gather_body(table, srcv, out, idx_v, rows_v, gsem):
    wid = lax.axis_index("s") * 2 + lax.axis_index("c")

    @pl.loop(0, NCH)
    def _chunk(ci):
        irow = wid * (EW // 128) + ci * 8
        pltpu.sync_copy(srcv.at[pl.ds(irow, 8)], idx_v)
        for half in range(2):
            descs = []
            for j in range(4):
                descs.append(
                    pltpu.async_copy(table.at[idx_v.at[half * 4 + j]],
                                     rows_v.at[pl.ds(j * 128, 128)], gsem))
            for d in descs:
                d.wait()
            base = wid * EW + ci * CH + half * CHG
            pltpu.sync_copy(rows_v, out.at[pl.ds(base, CHG)])


def _sc_gather(table, srcv):
    return pl.kernel(
        _gather_body,
        out_type=jax.ShapeDtypeStruct((EP, 128), jnp.float32),
        mesh=plsc.VectorSubcoreMesh(**_SC_MESH),
        scratch_types=[
            pltpu.VMEM((8, 128), jnp.int32),
            pltpu.VMEM((CHG, 128), jnp.float32),
            pltpu.SemaphoreType.DMA,
        ],
    )(table, srcv)


CHS = 256


def _scatter_body(msgv, dstv, zer, out, idx_v, rows_v, acc):
    cid = lax.axis_index("c")
    sid = lax.axis_index("s")
    wid = sid * 2 + cid

    pltpu.sync_copy(zer, acc.at[pl.ds(sid * NSUB, NSUB)])
    plsc.subcore_barrier()

    @pl.loop(0, NCH)
    def _chunk(ci):
        irow = wid * (EW // 128) + ci * 8
        pltpu.sync_copy(dstv.at[pl.ds(irow, 8)], idx_v)
        for q in range(4):
            base = wid * EW + ci * CH + q * CHS
            pltpu.sync_copy(msgv.at[pl.ds(base, CHS)], rows_v)
            for j in range(2):
                pltpu.sync_copy(rows_v.at[pl.ds(j * 128, 128)],
                                acc.at[idx_v.at[q * 2 + j]], add=True)

    plsc.subcore_barrier()
    pltpu.sync_copy(acc.at[pl.ds(sid * NSUB, NSUB)],
                    out.at[cid].at[pl.ds(sid * NSUB, NSUB)])


def _sc_scatter(msgv, dstv, zer):
    return pl.kernel(
        _scatter_body,
        out_type=jax.ShapeDtypeStruct((2, NP, 128), jnp.float32),
        mesh=plsc.VectorSubcoreMesh(**_SC_MESH),
        scratch_types=[
            pltpu.VMEM((8, 128), jnp.int32),
            pltpu.VMEM((CHS, 128), jnp.float32),
            pltpu.VMEM_SHARED((NP, 128), jnp.float32),
        ],
    )(msgv, dstv, zer)


_BE = 2048


def _dense_body(out_D, F_in, xs_ref, th_ref, pc_ref, selA_ref, selB_ref, S_ref,
                wf_ref, u0_ref, u1_ref, v0_ref, v1_ref, msg_ref):
    B = xs_ref.shape[0]
    xs = xs_ref[:, :F_in]
    th = th_ref[...]
    c1 = jnp.cos(th)
    s1 = jnp.sin(th)
    c2 = 2.0 * c1 * c1 - 1.0
    s2 = 2.0 * s1 * c1
    one = jnp.ones_like(th)
    trigA = jnp.concatenate([one, c1, c2], axis=1)
    trigB = jnp.concatenate([s1, s2], axis=1)
    dn = (((1,), (0,)), ((), ()))
    A = lax.dot_general(trigA, selA_ref[...], dn, preferred_element_type=jnp.float32)
    Bm = lax.dot_general(trigB, selB_ref[...], dn, preferred_element_type=jnp.float32)
    xsw = lax.dot_general(xs, S_ref[...], dn, preferred_element_type=jnp.float32)
    rot = xs * A + xsw * Bm
    T = lax.dot_general(rot, wf_ref[...], dn, preferred_element_type=jnp.float32)
    P = pc_ref[...]
    if out_D == 5:
        msg = (lax.dot_general(T, u0_ref[...], dn, preferred_element_type=jnp.float32)
               * lax.dot_general(P, v0_ref[...], dn, preferred_element_type=jnp.float32)
               + lax.dot_general(T, u1_ref[...], dn, preferred_element_type=jnp.float32)
               * lax.dot_general(P, v1_ref[...], dn, preferred_element_type=jnp.float32))
    else:
        msg = T[:, :16] * P[:, 0:1] + T[:, 16:32] * P[:, 5:6]
    eid = pl.program_id(0) * B + lax.broadcasted_iota(jnp.int32, (B, 1), 0)
    F_out = msg.shape[1]
    msg_ref[:, :F_out] = jnp.where(eid < E, msg, 0.0)
    msg_ref[:, F_out:] = jnp.zeros((B, 128 - F_out), jnp.float32)


def _dense_body_d1(xs_ref, pc_ref, wf_ref, msg_ref):
    B = xs_ref.shape[0]
    dn = (((1,), (0,)), ((), ()))
    T = lax.dot_general(xs_ref[:, :16], wf_ref[...], dn,
                        preferred_element_type=jnp.float32)
    P = pc_ref[...]
    msg = T[:, :16] * P[:, 0:1] + T[:, 16:32] * P[:, 5:6]
    eid = pl.program_id(0) * B + lax.broadcasted_iota(jnp.int32, (B, 1), 0)
    msg_ref[:, :16] = jnp.where(eid < E, msg, 0.0)
    msg_ref[:, 16:] = jnp.zeros((B, 112), jnp.float32)


def _uv_consts():
    u0 = np.zeros((32, 80), np.float32)
    u1 = np.zeros((32, 80), np.float32)
    v0 = np.zeros((10, 80), np.float32)
    v1 = np.zeros((10, 80), np.float32)
    for o in range(16):
        for p in range(5):
            u0[o, o * 5 + p] = 1.0
            u1[16 + o, o * 5 + p] = 1.0
            v0[p, o * 5 + p] = 1.0
            v1[5 + p, o * 5 + p] = 1.0
    return tuple(jnp.asarray(a) for a in (u0, u1, v0, v1))


def _sel_consts(C_in, F_in):
    selA = np.zeros((3, F_in), np.float32)
    selB = np.zeros((2, F_in), np.float32)
    S = np.zeros((F_in, F_in), np.float32)
    for i in range(C_in):
        o = i * 5
        selA[0, o] = 1.0
        selA[1, o + 1] = selA[1, o + 2] = 1.0
        selA[2, o + 3] = selA[2, o + 4] = 1.0
        selB[0, o + 1] = -1.0
        selB[0, o + 2] = 1.0
        selB[1, o + 3] = -1.0
        selB[1, o + 4] = 1.0
        S[o + 2, o + 1] = 1.0
        S[o + 1, o + 2] = 1.0
        S[o + 4, o + 3] = 1.0
        S[o + 3, o + 4] = 1.0
    return tuple(jnp.asarray(a) for a in (selA, selB, S))


def _tc_dense(xs, th, pc, wf, C_in, F_in, out_D):
    grid = EP // _BE
    eb = lambda c: pl.BlockSpec((_BE, c), lambda i: (i, 0))
    wb = lambda a: pl.BlockSpec(a.shape, lambda i: (0, 0))
    cs = _sel_consts(C_in, F_in) + (wf,) + _uv_consts()
    return pl.pallas_call(
        functools.partial(_dense_body, out_D, F_in),
        grid=(grid,),
        in_specs=[eb(128), eb(1), eb(10)] + [wb(a) for a in cs],
        out_specs=eb(128),
        out_shape=jax.ShapeDtypeStruct((EP, 128), jnp.float32),
    )(xs, th, pc, *cs)


def _tc_dense_d1(xs, pc, wf):
    grid = EP // _BE
    eb = lambda c: pl.BlockSpec((_BE, c), lambda i: (i, 0))
    return pl.pallas_call(
        _dense_body_d1,
        grid=(grid,),
        in_specs=[eb(128), eb(10), pl.BlockSpec(wf.shape, lambda i: (0, 0))],
        out_specs=eb(128),
        out_shape=jax.ShapeDtypeStruct((EP, 128), jnp.float32),
    )(xs, pc, wf)


def _epi_body(F, acc_ref, b_ref, sc_ref, m_ref, h_ref):
    h = acc_ref[0, :N, :F] + acc_ref[1, :N, :F] + b_ref[...]
    if m_ref is not None:
        dn = (((1,), (0,)), ((), ()))
        h = h + lax.dot_general(sc_ref[...], m_ref[...], dn,
                                preferred_element_type=jnp.float32)
    elif sc_ref is not None:
        h = h + sc_ref[:, :F]
    h = jnp.maximum(h, 0.0)
    h_ref[:, :F] = h
    h_ref[:, F:] = jnp.zeros((N, 128 - F), jnp.float32)


def _tc_epilogue(acc, bias, F, sc=None, m=None):
    args = [acc, bias.reshape(1, F)]
    if sc is not None:
        args.append(sc)
    if m is not None:
        args.append(m)

    def body(*refs):
        acc_ref, b_ref = refs[0], refs[1]
        sc_ref = refs[2] if sc is not None else None
        m_ref = refs[3] if m is not None else None
        _epi_body(F, acc_ref, b_ref, sc_ref, m_ref, refs[-1])

    return pl.pallas_call(
        body,
        out_shape=jax.ShapeDtypeStruct((N, 128), jnp.float32),
    )(*args)


def _final_body(acc_ref, b_ref, xb_ref, m_ref, w1_ref, b1_ref, w2_ref,
                b2_ref, batch_ref, out_ref):
    dn = (((1,), (0,)), ((), ()))
    dnt = (((1,), (1,)), ((), ()))
    h = acc_ref[0, :N, :16] + acc_ref[1, :N, :16] + b_ref[...]
    h = h + lax.dot_general(xb_ref[...], m_ref[...], dn,
                            preferred_element_type=jnp.float32)
    h = jnp.maximum(h, 0.0)
    h1 = jnp.maximum(
        lax.dot_general(h, w1_ref[...], dnt,
                        preferred_element_type=jnp.float32) + b1_ref[...], 0.0)
    h2 = lax.dot_general(h1, w2_ref[...], dnt,
                         preferred_element_type=jnp.float32) + b2_ref[...]
    seg = lax.broadcasted_iota(jnp.int32, (NG, N), 0)
    onehot = jnp.where(batch_ref[...] == seg, 1.0, 0.0)
    sums = lax.dot_general(onehot, h2, dn, preferred_element_type=jnp.float32)
    cnt = jnp.sum(onehot, axis=1, keepdims=True)
    pooled = sums / jnp.maximum(cnt, 1.0)
    mx = jnp.max(pooled, axis=1, keepdims=True)
    lse = mx + jnp.log(jnp.sum(jnp.exp(pooled - mx), axis=1, keepdims=True))
    out_ref[...] = pooled - lse


def _tc_final(acc, b16, xb3, m3, Wl1, bl1, Wl2, bl2, batch):
    return pl.pallas_call(
        _final_body,
        out_shape=jax.ShapeDtypeStruct((NG, 40), jnp.float32),
    )(acc, b16.reshape(1, 16), xb3, m3, Wl1, bl1.reshape(1, 256), Wl2,
      bl2.reshape(1, 40), batch.reshape(1, N).astype(jnp.int32))


def _rot_basis(C_in, F_in):
    M = [np.zeros((F_in, F_in), np.float32) for _ in range(5)]
    for i in range(C_in):
        o = i * 5
        M[0][o, o] = 1.0
        M[1][o + 1, o + 1] = M[1][o + 2, o + 2] = 1.0
        M[2][o + 2, o + 1] = -1.0
        M[2][o + 1, o + 2] = 1.0
        M[3][o + 3, o + 3] = M[3][o + 4, o + 4] = 1.0
        M[4][o + 4, o + 3] = -1.0
        M[4][o + 3, o + 4] = 1.0
    return M


def _wk(W, C_in, F_in):
    wf = _wflat(W, F_in)
    M = _rot_basis(C_in, F_in)
    return jnp.concatenate([jnp.asarray(m) @ wf for m in M], axis=1)


def _wflat(W, F_in):
    NR, CO, CI, D = W.shape
    wf = jnp.transpose(W, (2, 3, 0, 1)).reshape(CI * D, NR * CO)
    if CI * D < F_in:
        wf = jnp.pad(wf, ((0, F_in - CI * D), (0, 0)))
    return wf.astype(jnp.float32)


def _shortcut_mat(Ws, D):
    CO, CI = Ws.shape
    m = jnp.zeros((CI * 5, CO * D), jnp.float32)
    for p in range(D):
        m = m.at[p::5, p::D].set(jnp.transpose(Ws))
    return m


def kernel(x, edge_index, precomp, connection, batch, W1a, b1a, W1b, b1b, Ws1,
           W2a, b2a, W2b, b2b, W3a, b3a, W3b, b3b, Ws3, Wl1, bl1, Wl2, bl2):
    f32 = jnp.float32
    pad = EP - E
    fill = (jnp.arange(pad, dtype=jnp.int32) % N)
    srcv = jnp.concatenate([edge_index[0].astype(jnp.int32), fill]).reshape(EP // 128, 128)
    dstv = jnp.concatenate([edge_index[1].astype(jnp.int32), fill]).reshape(EP // 128, 128)
    th = jnp.pad(connection.astype(f32), (0, pad)).reshape(EP, 1)
    pc = jnp.pad(precomp.astype(f32).reshape(E, 10), ((0, pad), (0, 0)))

    z128 = jnp.zeros((NSUB, 128), f32)

    def conv(table, C_in, F_in, wf, out_D):
        xs = _sc_gather(table, srcv)
        if wf.shape[1] == 32 and F_in == 16:
            msg = _tc_dense_d1(xs, pc, wf)
        else:
            msg = _tc_dense(xs, th, pc, wf, C_in, F_in, out_D)
        return _sc_scatter(msg, dstv, z128)

    def bias80(b):
        return jnp.repeat(b.astype(f32), 5) * jnp.tile(jnp.asarray([1., 0, 0, 0, 0]), 16)

    t0 = jnp.pad(x.astype(f32).reshape(N, 35), ((0, 0), (0, 93)))

    a = conv(t0, 7, 40, _wflat(W1a, 40), 5)
    h1 = _tc_epilogue(a, bias80(b1a), 80)
    a = conv(h1, 16, 80, _wflat(W1b, 80), 5)
    hb1 = _tc_epilogue(a, bias80(b1b), 80, sc=t0,
                       m=jnp.pad(_shortcut_mat(Ws1, 5), ((0, 93), (0, 0))))

    a = conv(hb1, 16, 80, _wflat(W2a, 80), 5)
    h2 = _tc_epilogue(a, bias80(b2a), 80)
    a = conv(h2, 16, 80, _wflat(W2b, 80), 5)
    hb2 = _tc_epilogue(a, bias80(b2b), 80, sc=hb1)

    a = conv(hb2, 16, 80, _wflat(W3a, 80), 1)
    h3 = _tc_epilogue(a, b3a.astype(f32), 16)
    a = conv(h3, 16, 16, _wflat(W3b, 16), 1)

    m3 = jnp.pad(_shortcut_mat(Ws3, 1), ((0, 48), (0, 0)))
    return _tc_final(a, b3b.astype(f32), hb2, m3, Wl1.astype(f32), bl1,
                     Wl2.astype(f32), bl2, batch)
